# baseline (speedup 1.0000x reference)
import functools

import jax
import jax.numpy as jnp
from jax import lax
from jax.experimental import pallas as pl
from jax.experimental.pallas import tpu as pltpu
from jax.experimental.pallas import tpu_sc as plsc

N = 10000
E = 640000
QA_DIM = 1024
HID = 128
N_NTYPE = 4
N_ETYPE = 38
N_GRAPHS = 50
GC_IN = HID + HID // 2
EA_DIM = N_NTYPE + N_ETYPE + N_NTYPE

HEXT = HID
NC, NS, L = 2, 16, 16
NW = NC * NS
RC = 32
KB = 10
NSB = 64
NCHUNK = NSB * KB
EPW = NCHUNK * RC
E_PAD = NW * EPW
NB = 1000
NBLK = N // NB
EB = 2000
EBLK = E // EB
N_ACC = 10240
PB = 1024
PBLK = N_ACC // PB
KBRC = KB * RC
RPT = 624


def _node_body(x_ref, nt_ref, ns_ref, qa_ref, wqa_ref, bqa_ref, wnt_ref, bnt_ref,
               wlin_ref, asrc_w_ref, adst_w_ref, hext_ref, ad_ref):
    i = pl.program_id(0)
    x = x_ref[...]
    qa_row = jnp.dot(qa_ref[...], wqa_ref[...],
                     preferred_element_type=jnp.float32) + bqa_ref[...]
    row_ids = lax.broadcasted_iota(jnp.int32, (NB, 1), 0) + i * NB
    x = jnp.where(row_ids == 0, qa_row, x)
    xr = jax.nn.relu(x)
    nts = jnp.concatenate([nt_ref[...], ns_ref[...]], axis=-1)
    extras = jnp.dot(nts, wnt_ref[...], preferred_element_type=jnp.float32) + bnt_ref[...]
    er = jax.nn.relu(extras)
    h = (jnp.dot(xr, wlin_ref[0:HID, :], preferred_element_type=jnp.float32)
         + jnp.dot(er, wlin_ref[HID:GC_IN, :], preferred_element_type=jnp.float32))
    hext_ref[...] = h
    a_src = jnp.sum(h * asrc_w_ref[...], axis=-1)
    a_dst = jnp.sum(h * adst_w_ref[...], axis=-1)
    ad_ref[0, 0, :] = a_src
    ad_ref[0, 1, :] = a_dst


def _node_stage(x, node_types, node_scores, qa_emb, W_qa, b_qa, W_nt, b_nt, W_lin,
                att_src, att_dst):
    full = lambda shape: pl.BlockSpec(shape, lambda i: (0,) * len(shape))
    return pl.pallas_call(
        _node_body,
        grid=(NBLK,),
        in_specs=[
            pl.BlockSpec((NB, HID), lambda i: (i, 0)),
            pl.BlockSpec((NB, N_NTYPE), lambda i: (i, 0)),
            pl.BlockSpec((NB, 1), lambda i: (i, 0)),
            full((1, QA_DIM)),
            full((QA_DIM, HID)),
            full((1, HID)),
            full((N_NTYPE + 1, HID // 2)),
            full((1, HID // 2)),
            full((GC_IN, HID)),
            full((1, HID)),
            full((1, HID)),
        ],
        out_specs=[
            pl.BlockSpec((NB, HEXT), lambda i: (i, 0)),
            pl.BlockSpec((1, 2, NB), lambda i: (i, 0, 0)),
        ],
        out_shape=[
            jax.ShapeDtypeStruct((N, HEXT), jnp.float32),
            jax.ShapeDtypeStruct((NBLK, 2, NB), jnp.float32),
        ],
    )(x, node_types, node_scores, qa_emb.reshape(1, QA_DIM), W_qa,
      b_qa.reshape(1, HID), W_nt, b_nt.reshape(1, HID // 2), W_lin,
      att_src.reshape(1, HID), att_dst.reshape(1, HID))


def _edge_body(ea_ref, we1_ref, be1_ref, we2_ref, be2_ref, wel_ref, atte_ref, out_ref):
    t = jax.nn.relu(jnp.dot(ea_ref[...], we1_ref[...],
                            preferred_element_type=jnp.float32) + be1_ref[...])
    s = jax.nn.relu(jnp.dot(t, we2_ref[...],
                            preferred_element_type=jnp.float32) + be2_ref[...])
    v_el = jnp.dot(wel_ref[...], atte_ref[...], preferred_element_type=jnp.float32)
    out_ref[...] = jnp.dot(s, v_el, preferred_element_type=jnp.float32).reshape(1, 1, EB)


def _edge_stage(edge_attr, W_e1, b_e1, W_e2, b_e2, W_el, att_edge):
    full = lambda shape: pl.BlockSpec(shape, lambda i: (0,) * len(shape))
    return pl.pallas_call(
        _edge_body,
        grid=(EBLK,),
        in_specs=[
            pl.BlockSpec((EB, EA_DIM), lambda i: (i, 0)),
            full((EA_DIM, HID)),
            full((1, HID)),
            full((HID, HID)),
            full((1, HID)),
            full((HID, HID)),
            full((HID, 1)),
        ],
        out_specs=pl.BlockSpec((1, 1, EB), lambda i: (i, 0, 0)),
        out_shape=jax.ShapeDtypeStruct((EBLK, 1, EB), jnp.float32),
    )(edge_attr, W_e1, b_e1.reshape(1, HID), W_e2, b_e2.reshape(1, HID),
      W_el, att_edge.reshape(HID, 1))


def _sc_body(src_hbm, dst_hbm, ae_hbm, asrc_hbm, adst_hbm, hext_hbm, zeros_hbm,
             out_hbm, den_hbm, asrc_v, adst_v, srcc_v, dstc_v, aec_v, dr0_v, dr1_v,
             ex_v, rows_v, den_v, acc_sh, gsem0, gsem1, ssem0, ssem1, isem0, isem1):
    cid = lax.axis_index("c")
    sid = lax.axis_index("s")
    wid = cid * NS + sid
    gsem = (gsem0, gsem1)
    ssem = (ssem0, ssem1)
    isem = (isem0, isem1)
    drow = (dr0_v, dr1_v)
    idx_bufs = (srcc_v, dstc_v, aec_v)
    idx_hbms = (src_hbm, dst_hbm, ae_hbm)

    pltpu.sync_copy(asrc_hbm, asrc_v)
    pltpu.sync_copy(adst_hbm, adst_v)

    @pl.when(sid < NS - 1)
    def _():
        pltpu.sync_copy(zeros_hbm.at[pl.ds(sid * RPT, RPT)],
                        acc_sh.at[pl.ds(sid * RPT, RPT)])

    @pl.when(sid == NS - 1)
    def _():
        pltpu.sync_copy(zeros_hbm.at[pl.ds((NS - 1) * RPT, N - (NS - 1) * RPT)],
                        acc_sh.at[pl.ds((NS - 1) * RPT, N - (NS - 1) * RPT)])

    def zero_body(i, c):
        den_v[pl.ds(i * L, L)] = jnp.zeros((L,), jnp.float32)
        return c

    lax.fori_loop(0, N_ACC // L, zero_body, 0)
    plsc.subcore_barrier()

    ebase = wid * EPW

    def stage(S, sb):
        for hbm, buf in zip(idx_hbms, idx_bufs):
            pltpu.async_copy(hbm.at[pl.ds(ebase + S * KBRC, KBRC)],
                             buf.at[pl.ds(sb * KBRC, KBRC)], isem[sb])

    def stage_wait(sb):
        for hbm, buf in zip(idx_hbms, idx_bufs):
            pltpu.make_async_copy(hbm.at[pl.ds(0, KBRC)],
                                  buf.at[pl.ds(sb * KBRC, KBRC)],
                                  isem[sb]).wait()

    def gather(sb, kk, b):
        pltpu.async_copy(
            hext_hbm.at[srcc_v.at[pl.ds(sb * KBRC + kk * RC, RC)]],
            rows_v.at[b], gsem[b])

    def gather_wait(b):
        pltpu.make_async_copy(hext_hbm.at[srcc_v.at[pl.ds(0, RC)]],
                              rows_v.at[b], gsem[b]).wait()

    def scatter(b):
        pltpu.async_copy(rows_v.at[b], acc_sh.at[drow[b]], ssem[b], add=True)

    def scatter_wait(b):
        pltpu.make_async_copy(rows_v.at[0], acc_sh.at[dr0_v], ssem[b]).wait()

    stage(0, 0)
    stage_wait(0)
    gather(0, 0, 0)

    def pair_body(S2, carry):
        for sp in (0, 1):
            S = S2 * 2 + sp
            for kk in range(KB):
                b = kk % 2
                b1 = 1 - b
                if kk == 0:
                    @pl.when(S >= 1)
                    def _():
                        scatter_wait(b1)
                else:
                    scatter_wait(b1)
                if kk < KB - 1:
                    gather(sp, kk + 1, b1)
                else:
                    @pl.when(S < NSB - 1)
                    def _():
                        stage_wait(1 - sp)
                        gather(1 - sp, 0, b1)
                gather_wait(b)
                cb = sp * KBRC + kk * RC
                for g in range(RC // L):
                    idx_s = srcc_v[pl.ds(cb + g * L, L)]
                    idx_d = dstc_v[pl.ds(cb + g * L, L)]
                    drow[b][pl.ds(g * L, L)] = idx_d
                    a_s = plsc.load_gather(asrc_v, [idx_s])
                    a_d = plsc.load_gather(adst_v, [idx_d])
                    al = a_s + a_d + aec_v[pl.ds(cb + g * L, L)]
                    al = jnp.where(al >= 0.0, al, al * 0.2)
                    ex = jnp.exp(al)
                    ex_v[pl.ds(g * L, L)] = ex
                    plsc.addupdate_scatter(den_v, [idx_d], ex)

                @plsc.parallel_loop(0, RC, unroll=4)
                def scale_body(e):
                    exb = plsc.load_gather(ex_v, [jnp.broadcast_to(e, (L,))])
                    for k in range(HID // L):
                        rows_v[b, e, pl.ds(k * L, L)] = (
                            rows_v[b, e, pl.ds(k * L, L)] * exb)

                if kk == 2:
                    @pl.when(S < NSB - 1)
                    def _():
                        stage(S + 1, 1 - sp)
                scatter(b)
        return carry

    lax.fori_loop(0, NSB // 2, pair_body, 0)
    scatter_wait((KB - 1) % 2)
    pltpu.sync_copy(den_v, den_hbm.at[wid])
    plsc.subcore_barrier()

    @pl.when(sid < NS - 1)
    def _():
        pltpu.sync_copy(acc_sh.at[pl.ds(sid * RPT, RPT)],
                        out_hbm.at[cid, pl.ds(sid * RPT, RPT)])

    @pl.when(sid == NS - 1)
    def _():
        pltpu.sync_copy(acc_sh.at[pl.ds((NS - 1) * RPT, N - (NS - 1) * RPT)],
                        out_hbm.at[cid, pl.ds((NS - 1) * RPT, N - (NS - 1) * RPT)])


@functools.cache
def _sc_gat():
    mesh = plsc.VectorSubcoreMesh(core_axis_name="c", subcore_axis_name="s",
                                  num_cores=NC, num_subcores=NS)
    return pl.kernel(
        _sc_body,
        out_type=(jax.ShapeDtypeStruct((NC, N_ACC, HEXT), jnp.float32),
                  jax.ShapeDtypeStruct((NW, N_ACC), jnp.float32)),
        mesh=mesh,
        compiler_params=pltpu.CompilerParams(needs_layout_passes=False),
        scratch_types=[
            pltpu.VMEM((N,), jnp.float32),
            pltpu.VMEM((N,), jnp.float32),
            pltpu.VMEM((2 * KBRC,), jnp.int32),
            pltpu.VMEM((2 * KBRC,), jnp.int32),
            pltpu.VMEM((2 * KBRC,), jnp.float32),
            pltpu.VMEM((RC,), jnp.int32),
            pltpu.VMEM((RC,), jnp.int32),
            pltpu.VMEM((RC,), jnp.float32),
            pltpu.VMEM((2, RC, HEXT), jnp.float32),
            pltpu.VMEM((N_ACC,), jnp.float32),
            pltpu.VMEM_SHARED((N, HEXT), jnp.float32),
            pltpu.SemaphoreType.DMA,
            pltpu.SemaphoreType.DMA,
            pltpu.SemaphoreType.DMA,
            pltpu.SemaphoreType.DMA,
            pltpu.SemaphoreType.DMA,
            pltpu.SemaphoreType.DMA,
        ],
    )


def _pool_body(acc_ref, den_ref, n2g_ref, bgat_ref, out0_ref, p_ref, sums_sc, cnt_sc):
    i = pl.program_id(0)
    num = acc_ref[0] + acc_ref[1]
    den = jnp.sum(den_ref[:, pl.ds(i * PB, PB)], axis=0)[:, None]
    out = jnp.where(den > 0.0, num / (den + 1e-16), 0.0) + bgat_ref[...]

    @pl.when(i == 0)
    def _():
        out0_ref[...] = out[0:1, :]
        sums_sc[...] = jnp.zeros_like(sums_sc)
        cnt_sc[...] = jnp.zeros_like(cnt_sc)

    n2g = n2g_ref[0, 0, :]
    ind = (lax.broadcasted_iota(jnp.int32, (N_GRAPHS, PB), 0)
           == n2g[None, :]).astype(jnp.float32)
    sums_sc[...] += jnp.dot(ind, out, preferred_element_type=jnp.float32)
    cnt_sc[...] += jnp.broadcast_to(jnp.sum(ind, axis=-1)[:, None], (N_GRAPHS, HID))

    @pl.when(i == PBLK - 1)
    def _():
        p_ref[...] = sums_sc[...] / jnp.maximum(cnt_sc[...], 1.0)


def _pool_stage(acc, den, node2graph, b_gat):
    full = lambda shape: pl.BlockSpec(shape, lambda i: (0,) * len(shape))
    n2g = jnp.concatenate(
        [node2graph.astype(jnp.int32),
         jnp.full((N_ACC - N,), N_GRAPHS, jnp.int32)]).reshape(PBLK, 1, PB)
    return pl.pallas_call(
        _pool_body,
        grid=(PBLK,),
        in_specs=[
            pl.BlockSpec((NC, PB, HEXT), lambda i: (0, i, 0)),
            pl.BlockSpec((NW, N_ACC), lambda i: (0, 0)),
            pl.BlockSpec((1, 1, PB), lambda i: (i, 0, 0)),
            full((1, HID)),
        ],
        out_specs=[
            full((1, HID)),
            full((N_GRAPHS, HID)),
        ],
        out_shape=[
            jax.ShapeDtypeStruct((1, HID), jnp.float32),
            jax.ShapeDtypeStruct((N_GRAPHS, HID), jnp.float32),
        ],
        scratch_shapes=[
            pltpu.VMEM((N_GRAPHS, HID), jnp.float32),
            pltpu.VMEM((N_GRAPHS, HID), jnp.float32),
        ],
    )(acc, den, n2g, b_gat.reshape(1, HID))


def kernel(qa_emb, x, node_ids, node_types, node_scores, edge_index, edge_type,
           edge_attr, node2graph, W_qa, b_qa, W_nt, b_nt, W_e1, b_e1, W_e2, b_e2,
           W_lin, W_el, att_src, att_dst, att_edge, b_gat):
    h_ext, ad = _node_stage(x, node_types, node_scores, qa_emb, W_qa, b_qa,
                            W_nt, b_nt, W_lin, att_src, att_dst)
    a_src = ad[:, 0, :].reshape(N)
    a_dst = ad[:, 1, :].reshape(N)
    alpha_e = _edge_stage(edge_attr, W_e1, b_e1, W_e2, b_e2, W_el,
                          att_edge).reshape(E)
    pad = E_PAD - E
    src = jnp.concatenate(
        [edge_index[0].astype(jnp.int32),
         jnp.zeros((pad,), jnp.int32)])
    dst = jnp.concatenate(
        [edge_index[1].astype(jnp.int32),
         jnp.zeros((pad,), jnp.int32)])
    ae2 = jnp.concatenate(
        [alpha_e, jnp.full((pad,), -1e30, jnp.float32)])
    zeros = jnp.zeros((N, HEXT), jnp.float32)
    acc, den = _sc_gat()(src, dst, ae2, a_src, a_dst, h_ext, zeros)
    out0, p = _pool_stage(acc, den, node2graph, b_gat)
    return (out0.reshape(HID), p)


# speedup vs baseline: 1.0257x; 1.0257x over previous
import functools

import jax
import jax.numpy as jnp
from jax import lax
from jax.experimental import pallas as pl
from jax.experimental.pallas import tpu as pltpu
from jax.experimental.pallas import tpu_sc as plsc

N = 10000
E = 640000
QA_DIM = 1024
HID = 128
N_NTYPE = 4
N_ETYPE = 38
N_GRAPHS = 50
GC_IN = HID + HID // 2
EA_DIM = N_NTYPE + N_ETYPE + N_NTYPE

HEXT = HID
NC, NS, L = 2, 16, 16
NW = NC * NS
RC = 32
KB = 8
NSB = 80
NCHUNK = NSB * KB
EPW = NCHUNK * RC
E_PAD = NW * EPW
NB = 1000
NBLK = N // NB
EB = 2000
EBLK = E // EB
N_ACC = 10240
PB = 1024
PBLK = N_ACC // PB
KBRC = KB * RC
RPT = 624


def _node_body(x_ref, nt_ref, ns_ref, qa_ref, wqa_ref, bqa_ref, wnt_ref, bnt_ref,
               wlin_ref, asrc_w_ref, adst_w_ref, hext_ref, ad_ref):
    i = pl.program_id(0)
    x = x_ref[...]
    qa_row = jnp.dot(qa_ref[...], wqa_ref[...],
                     preferred_element_type=jnp.float32) + bqa_ref[...]
    row_ids = lax.broadcasted_iota(jnp.int32, (NB, 1), 0) + i * NB
    x = jnp.where(row_ids == 0, qa_row, x)
    xr = jax.nn.relu(x)
    nts = jnp.concatenate([nt_ref[...], ns_ref[...]], axis=-1)
    extras = jnp.dot(nts, wnt_ref[...], preferred_element_type=jnp.float32) + bnt_ref[...]
    er = jax.nn.relu(extras)
    h = (jnp.dot(xr, wlin_ref[0:HID, :], preferred_element_type=jnp.float32)
         + jnp.dot(er, wlin_ref[HID:GC_IN, :], preferred_element_type=jnp.float32))
    hext_ref[...] = h
    a_src = jnp.sum(h * asrc_w_ref[...], axis=-1)
    a_dst = jnp.sum(h * adst_w_ref[...], axis=-1)
    ad_ref[0, 0, :] = a_src
    ad_ref[0, 1, :] = a_dst


def _node_stage(x, node_types, node_scores, qa_emb, W_qa, b_qa, W_nt, b_nt, W_lin,
                att_src, att_dst):
    full = lambda shape: pl.BlockSpec(shape, lambda i: (0,) * len(shape))
    return pl.pallas_call(
        _node_body,
        grid=(NBLK,),
        in_specs=[
            pl.BlockSpec((NB, HID), lambda i: (i, 0)),
            pl.BlockSpec((NB, N_NTYPE), lambda i: (i, 0)),
            pl.BlockSpec((NB, 1), lambda i: (i, 0)),
            full((1, QA_DIM)),
            full((QA_DIM, HID)),
            full((1, HID)),
            full((N_NTYPE + 1, HID // 2)),
            full((1, HID // 2)),
            full((GC_IN, HID)),
            full((1, HID)),
            full((1, HID)),
        ],
        out_specs=[
            pl.BlockSpec((NB, HEXT), lambda i: (i, 0)),
            pl.BlockSpec((1, 2, NB), lambda i: (i, 0, 0)),
        ],
        out_shape=[
            jax.ShapeDtypeStruct((N, HEXT), jnp.float32),
            jax.ShapeDtypeStruct((NBLK, 2, NB), jnp.float32),
        ],
    )(x, node_types, node_scores, qa_emb.reshape(1, QA_DIM), W_qa,
      b_qa.reshape(1, HID), W_nt, b_nt.reshape(1, HID // 2), W_lin,
      att_src.reshape(1, HID), att_dst.reshape(1, HID))


def _edge_body(ea_ref, we1_ref, be1_ref, we2_ref, be2_ref, wel_ref, atte_ref, out_ref):
    t = jax.nn.relu(jnp.dot(ea_ref[...], we1_ref[...],
                            preferred_element_type=jnp.float32) + be1_ref[...])
    s = jax.nn.relu(jnp.dot(t, we2_ref[...],
                            preferred_element_type=jnp.float32) + be2_ref[...])
    v_el = jnp.dot(wel_ref[...], atte_ref[...], preferred_element_type=jnp.float32)
    out_ref[...] = jnp.dot(s, v_el, preferred_element_type=jnp.float32).reshape(1, 1, EB)


def _edge_stage(edge_attr, W_e1, b_e1, W_e2, b_e2, W_el, att_edge):
    full = lambda shape: pl.BlockSpec(shape, lambda i: (0,) * len(shape))
    return pl.pallas_call(
        _edge_body,
        grid=(EBLK,),
        in_specs=[
            pl.BlockSpec((EB, EA_DIM), lambda i: (i, 0)),
            full((EA_DIM, HID)),
            full((1, HID)),
            full((HID, HID)),
            full((1, HID)),
            full((HID, HID)),
            full((HID, 1)),
        ],
        out_specs=pl.BlockSpec((1, 1, EB), lambda i: (i, 0, 0)),
        out_shape=jax.ShapeDtypeStruct((EBLK, 1, EB), jnp.float32),
    )(edge_attr, W_e1, b_e1.reshape(1, HID), W_e2, b_e2.reshape(1, HID),
      W_el, att_edge.reshape(HID, 1))


def _sc_body(src_hbm, dst_hbm, ae_hbm, asrc_hbm, adst_hbm, hext_hbm, zeros_hbm,
             out_hbm, den_hbm, asrc_v, adst_v, srcc_v, dstc_v, aec_v,
             dr0_v, dr1_v, dr2_v, dr3_v, ex_v, rows_v, den_v, acc_sh,
             gsem0, gsem1, gsem2, gsem3, ssem0, ssem1, ssem2, ssem3,
             isem0, isem1):
    cid = lax.axis_index("c")
    sid = lax.axis_index("s")
    wid = cid * NS + sid
    gsem = (gsem0, gsem1, gsem2, gsem3)
    ssem = (ssem0, ssem1, ssem2, ssem3)
    isem = (isem0, isem1)
    drow = (dr0_v, dr1_v, dr2_v, dr3_v)
    idx_bufs = (srcc_v, dstc_v, aec_v)
    idx_hbms = (src_hbm, dst_hbm, ae_hbm)

    pltpu.sync_copy(asrc_hbm, asrc_v)
    pltpu.sync_copy(adst_hbm, adst_v)

    @pl.when(sid < NS - 1)
    def _():
        pltpu.sync_copy(zeros_hbm.at[pl.ds(sid * RPT, RPT)],
                        acc_sh.at[pl.ds(sid * RPT, RPT)])

    @pl.when(sid == NS - 1)
    def _():
        pltpu.sync_copy(zeros_hbm.at[pl.ds((NS - 1) * RPT, N - (NS - 1) * RPT)],
                        acc_sh.at[pl.ds((NS - 1) * RPT, N - (NS - 1) * RPT)])

    def zero_body(i, c):
        den_v[pl.ds(i * L, L)] = jnp.zeros((L,), jnp.float32)
        return c

    lax.fori_loop(0, N_ACC // L, zero_body, 0)
    plsc.subcore_barrier()

    ebase = wid * EPW

    def stage(S, sb):
        for hbm, buf in zip(idx_hbms, idx_bufs):
            pltpu.async_copy(hbm.at[pl.ds(ebase + S * KBRC, KBRC)],
                             buf.at[pl.ds(sb * KBRC, KBRC)], isem[sb])

    def stage_wait(sb):
        for hbm, buf in zip(idx_hbms, idx_bufs):
            pltpu.make_async_copy(hbm.at[pl.ds(0, KBRC)],
                                  buf.at[pl.ds(sb * KBRC, KBRC)],
                                  isem[sb]).wait()

    def gather(sb, kk, b):
        pltpu.async_copy(
            hext_hbm.at[srcc_v.at[pl.ds(sb * KBRC + kk * RC, RC)]],
            rows_v.at[b], gsem[b])

    def gather_wait(b):
        pltpu.make_async_copy(hext_hbm.at[srcc_v.at[pl.ds(0, RC)]],
                              rows_v.at[b], gsem[b]).wait()

    def scatter(b):
        pltpu.async_copy(rows_v.at[b], acc_sh.at[drow[b]], ssem[b], add=True)

    def scatter_wait(b):
        pltpu.make_async_copy(rows_v.at[0], acc_sh.at[dr0_v], ssem[b]).wait()

    stage(0, 0)
    stage_wait(0)
    gather(0, 0, 0)
    gather(0, 1, 1)

    def pair_body(S2, carry):
        for sp in (0, 1):
            S = S2 * 2 + sp
            for kk in range(KB):
                b = kk % 4
                bn2 = (kk + 2) % 4
                if kk >= 2:
                    scatter_wait(bn2)
                else:
                    @pl.when(S >= 1)
                    def _():
                        scatter_wait(bn2)
                if kk < KB - 2:
                    gather(sp, kk + 2, bn2)
                elif kk == KB - 2:
                    @pl.when(S < NSB - 1)
                    def _():
                        stage_wait(1 - sp)
                        gather(1 - sp, 0, bn2)
                else:
                    @pl.when(S < NSB - 1)
                    def _():
                        gather(1 - sp, 1, bn2)
                gather_wait(b)
                cb = sp * KBRC + kk * RC
                for g in range(RC // L):
                    idx_s = srcc_v[pl.ds(cb + g * L, L)]
                    idx_d = dstc_v[pl.ds(cb + g * L, L)]
                    drow[b][pl.ds(g * L, L)] = idx_d
                    a_s = plsc.load_gather(asrc_v, [idx_s])
                    a_d = plsc.load_gather(adst_v, [idx_d])
                    al = a_s + a_d + aec_v[pl.ds(cb + g * L, L)]
                    al = jnp.where(al >= 0.0, al, al * 0.2)
                    ex = jnp.exp(al)
                    ex_v[pl.ds(g * L, L)] = ex
                    plsc.addupdate_scatter(den_v, [idx_d], ex)

                @plsc.parallel_loop(0, RC, unroll=4)
                def scale_body(e):
                    exb = plsc.load_gather(ex_v, [jnp.broadcast_to(e, (L,))])
                    for k in range(HID // L):
                        rows_v[b, e, pl.ds(k * L, L)] = (
                            rows_v[b, e, pl.ds(k * L, L)] * exb)

                if kk == 2:
                    @pl.when(S < NSB - 1)
                    def _():
                        stage(S + 1, 1 - sp)
                scatter(b)
        return carry

    lax.fori_loop(0, NSB // 2, pair_body, 0)
    scatter_wait((NCHUNK - 2) % 4)
    scatter_wait((NCHUNK - 1) % 4)
    pltpu.sync_copy(den_v, den_hbm.at[wid])
    plsc.subcore_barrier()

    @pl.when(sid < NS - 1)
    def _():
        pltpu.sync_copy(acc_sh.at[pl.ds(sid * RPT, RPT)],
                        out_hbm.at[cid, pl.ds(sid * RPT, RPT)])

    @pl.when(sid == NS - 1)
    def _():
        pltpu.sync_copy(acc_sh.at[pl.ds((NS - 1) * RPT, N - (NS - 1) * RPT)],
                        out_hbm.at[cid, pl.ds((NS - 1) * RPT, N - (NS - 1) * RPT)])


@functools.cache
def _sc_gat():
    mesh = plsc.VectorSubcoreMesh(core_axis_name="c", subcore_axis_name="s",
                                  num_cores=NC, num_subcores=NS)
    return pl.kernel(
        _sc_body,
        out_type=(jax.ShapeDtypeStruct((NC, N_ACC, HEXT), jnp.float32),
                  jax.ShapeDtypeStruct((NW, N_ACC), jnp.float32)),
        mesh=mesh,
        compiler_params=pltpu.CompilerParams(needs_layout_passes=False),
        scratch_types=[
            pltpu.VMEM((N,), jnp.float32),
            pltpu.VMEM((N,), jnp.float32),
            pltpu.VMEM((2 * KBRC,), jnp.int32),
            pltpu.VMEM((2 * KBRC,), jnp.int32),
            pltpu.VMEM((2 * KBRC,), jnp.float32),
            pltpu.VMEM((RC,), jnp.int32),
            pltpu.VMEM((RC,), jnp.int32),
            pltpu.VMEM((RC,), jnp.int32),
            pltpu.VMEM((RC,), jnp.int32),
            pltpu.VMEM((RC,), jnp.float32),
            pltpu.VMEM((4, RC, HEXT), jnp.float32),
            pltpu.VMEM((N_ACC,), jnp.float32),
            pltpu.VMEM_SHARED((N, HEXT), jnp.float32),
        ] + [pltpu.SemaphoreType.DMA] * 10,
    )


def _pool_body(acc_ref, den_ref, n2g_ref, bgat_ref, out0_ref, p_ref, sums_sc, cnt_sc):
    i = pl.program_id(0)
    num = acc_ref[0] + acc_ref[1]
    den = jnp.sum(den_ref[:, pl.ds(i * PB, PB)], axis=0)[:, None]
    out = jnp.where(den > 0.0, num / (den + 1e-16), 0.0) + bgat_ref[...]

    @pl.when(i == 0)
    def _():
        out0_ref[...] = out[0:1, :]
        sums_sc[...] = jnp.zeros_like(sums_sc)
        cnt_sc[...] = jnp.zeros_like(cnt_sc)

    n2g = n2g_ref[0, 0, :]
    ind = (lax.broadcasted_iota(jnp.int32, (N_GRAPHS, PB), 0)
           == n2g[None, :]).astype(jnp.float32)
    sums_sc[...] += jnp.dot(ind, out, preferred_element_type=jnp.float32)
    cnt_sc[...] += jnp.broadcast_to(jnp.sum(ind, axis=-1)[:, None], (N_GRAPHS, HID))

    @pl.when(i == PBLK - 1)
    def _():
        p_ref[...] = sums_sc[...] / jnp.maximum(cnt_sc[...], 1.0)


def _pool_stage(acc, den, node2graph, b_gat):
    full = lambda shape: pl.BlockSpec(shape, lambda i: (0,) * len(shape))
    n2g = jnp.concatenate(
        [node2graph.astype(jnp.int32),
         jnp.full((N_ACC - N,), N_GRAPHS, jnp.int32)]).reshape(PBLK, 1, PB)
    return pl.pallas_call(
        _pool_body,
        grid=(PBLK,),
        in_specs=[
            pl.BlockSpec((NC, PB, HEXT), lambda i: (0, i, 0)),
            pl.BlockSpec((NW, N_ACC), lambda i: (0, 0)),
            pl.BlockSpec((1, 1, PB), lambda i: (i, 0, 0)),
            full((1, HID)),
        ],
        out_specs=[
            full((1, HID)),
            full((N_GRAPHS, HID)),
        ],
        out_shape=[
            jax.ShapeDtypeStruct((1, HID), jnp.float32),
            jax.ShapeDtypeStruct((N_GRAPHS, HID), jnp.float32),
        ],
        scratch_shapes=[
            pltpu.VMEM((N_GRAPHS, HID), jnp.float32),
            pltpu.VMEM((N_GRAPHS, HID), jnp.float32),
        ],
    )(acc, den, n2g, b_gat.reshape(1, HID))


def kernel(qa_emb, x, node_ids, node_types, node_scores, edge_index, edge_type,
           edge_attr, node2graph, W_qa, b_qa, W_nt, b_nt, W_e1, b_e1, W_e2, b_e2,
           W_lin, W_el, att_src, att_dst, att_edge, b_gat):
    h_ext, ad = _node_stage(x, node_types, node_scores, qa_emb, W_qa, b_qa,
                            W_nt, b_nt, W_lin, att_src, att_dst)
    a_src = ad[:, 0, :].reshape(N)
    a_dst = ad[:, 1, :].reshape(N)
    alpha_e = _edge_stage(edge_attr, W_e1, b_e1, W_e2, b_e2, W_el,
                          att_edge).reshape(E)
    pad = E_PAD - E
    src = jnp.concatenate(
        [edge_index[0].astype(jnp.int32),
         jnp.zeros((pad,), jnp.int32)])
    dst = jnp.concatenate(
        [edge_index[1].astype(jnp.int32),
         jnp.zeros((pad,), jnp.int32)])
    ae2 = jnp.concatenate(
        [alpha_e, jnp.full((pad,), -1e30, jnp.float32)])
    zeros = jnp.zeros((N, HEXT), jnp.float32)
    acc, den = _sc_gat()(src, dst, ae2, a_src, a_dst, h_ext, zeros)
    out0, p = _pool_stage(acc, den, node2graph, b_gat)
    return (out0.reshape(HID), p)


# speedup vs baseline: 1.1008x; 1.0732x over previous
import functools

import jax
import jax.numpy as jnp
from jax import lax
from jax.experimental import pallas as pl
from jax.experimental.pallas import tpu as pltpu
from jax.experimental.pallas import tpu_sc as plsc

N = 10000
E = 640000
QA_DIM = 1024
HID = 128
N_NTYPE = 4
N_ETYPE = 38
N_GRAPHS = 50
GC_IN = HID + HID // 2
EA_DIM = N_NTYPE + N_ETYPE + N_NTYPE

HEXT = HID
NC, NS, L = 2, 16, 16
NW = NC * NS
RC = 32
KB = 8
NSB = 80
NCHUNK = NSB * KB
EPW = NCHUNK * RC
E_PAD = NW * EPW
NB = 1000
NBLK = N // NB
EB = 2000
EBLK = E // EB
N_ACC = 10240
PB = 1024
PBLK = N_ACC // PB
KBRC = KB * RC
RPT = 624


def _node_body(x_ref, nt_ref, ns_ref, qa_ref, wqa_ref, bqa_ref, wnt_ref, bnt_ref,
               wlin_ref, asrc_w_ref, adst_w_ref, hext_ref, ad_ref):
    i = pl.program_id(0)
    x = x_ref[...]
    qa_row = jnp.dot(qa_ref[...], wqa_ref[...],
                     preferred_element_type=jnp.float32) + bqa_ref[...]
    row_ids = lax.broadcasted_iota(jnp.int32, (NB, 1), 0) + i * NB
    x = jnp.where(row_ids == 0, qa_row, x)
    xr = jax.nn.relu(x)
    nts = jnp.concatenate([nt_ref[...], ns_ref[...]], axis=-1)
    extras = jnp.dot(nts, wnt_ref[...], preferred_element_type=jnp.float32) + bnt_ref[...]
    er = jax.nn.relu(extras)
    h = (jnp.dot(xr, wlin_ref[0:HID, :], preferred_element_type=jnp.float32)
         + jnp.dot(er, wlin_ref[HID:GC_IN, :], preferred_element_type=jnp.float32))
    hext_ref[...] = h
    a_src = jnp.sum(h * asrc_w_ref[...], axis=-1)
    a_dst = jnp.sum(h * adst_w_ref[...], axis=-1)
    ad_ref[0, 0, :] = a_src
    ad_ref[0, 1, :] = a_dst


def _node_stage(x, node_types, node_scores, qa_emb, W_qa, b_qa, W_nt, b_nt, W_lin,
                att_src, att_dst):
    full = lambda shape: pl.BlockSpec(shape, lambda i: (0,) * len(shape))
    return pl.pallas_call(
        _node_body,
        grid=(NBLK,),
        in_specs=[
            pl.BlockSpec((NB, HID), lambda i: (i, 0)),
            pl.BlockSpec((NB, N_NTYPE), lambda i: (i, 0)),
            pl.BlockSpec((NB, 1), lambda i: (i, 0)),
            full((1, QA_DIM)),
            full((QA_DIM, HID)),
            full((1, HID)),
            full((N_NTYPE + 1, HID // 2)),
            full((1, HID // 2)),
            full((GC_IN, HID)),
            full((1, HID)),
            full((1, HID)),
        ],
        out_specs=[
            pl.BlockSpec((NB, HEXT), lambda i: (i, 0)),
            pl.BlockSpec((1, 2, NB), lambda i: (i, 0, 0)),
        ],
        out_shape=[
            jax.ShapeDtypeStruct((N, HEXT), jnp.float32),
            jax.ShapeDtypeStruct((NBLK, 2, NB), jnp.float32),
        ],
    )(x, node_types, node_scores, qa_emb.reshape(1, QA_DIM), W_qa,
      b_qa.reshape(1, HID), W_nt, b_nt.reshape(1, HID // 2), W_lin,
      att_src.reshape(1, HID), att_dst.reshape(1, HID))


def _edge_body(ea_ref, we1_ref, be1_ref, we2_ref, be2_ref, wel_ref, atte_ref, out_ref):
    t = jax.nn.relu(jnp.dot(ea_ref[...], we1_ref[...],
                            preferred_element_type=jnp.float32) + be1_ref[...])
    s = jax.nn.relu(jnp.dot(t.astype(jnp.bfloat16), we2_ref[...],
                            preferred_element_type=jnp.float32) + be2_ref[...])
    v_el = jnp.dot(wel_ref[...].astype(jnp.float32), atte_ref[...],
                   preferred_element_type=jnp.float32)
    out_ref[...] = jnp.dot(s.astype(jnp.bfloat16), v_el.astype(jnp.bfloat16),
                           preferred_element_type=jnp.float32).reshape(1, 1, EB)


def _edge_stage(edge_attr, W_e1, b_e1, W_e2, b_e2, W_el, att_edge):
    full = lambda shape: pl.BlockSpec(shape, lambda i: (0,) * len(shape))
    return pl.pallas_call(
        _edge_body,
        grid=(EBLK,),
        in_specs=[
            pl.BlockSpec((EB, EA_DIM), lambda i: (i, 0)),
            full((EA_DIM, HID)),
            full((1, HID)),
            full((HID, HID)),
            full((1, HID)),
            full((HID, HID)),
            full((HID, 1)),
        ],
        out_specs=pl.BlockSpec((1, 1, EB), lambda i: (i, 0, 0)),
        out_shape=jax.ShapeDtypeStruct((EBLK, 1, EB), jnp.float32),
    )(edge_attr.astype(jnp.bfloat16), W_e1.astype(jnp.bfloat16),
      b_e1.reshape(1, HID), W_e2.astype(jnp.bfloat16), b_e2.reshape(1, HID),
      W_el.astype(jnp.bfloat16), att_edge.reshape(HID, 1))


def _sc_body(src_hbm, dst_hbm, ae_hbm, asrc_hbm, adst_hbm, hext_hbm, zeros_hbm,
             out_hbm, den_hbm, asrc_v, adst_v, srcc_v, dstc_v, aec_v,
             dr0_v, dr1_v, dr2_v, dr3_v, ex_v, rows_v, den_v, acc_sh,
             gsem0, gsem1, gsem2, gsem3, ssem0, ssem1, ssem2, ssem3,
             isem0, isem1):
    cid = lax.axis_index("c")
    sid = lax.axis_index("s")
    wid = cid * NS + sid
    gsem = (gsem0, gsem1, gsem2, gsem3)
    ssem = (ssem0, ssem1, ssem2, ssem3)
    isem = (isem0, isem1)
    drow = (dr0_v, dr1_v, dr2_v, dr3_v)
    idx_bufs = (srcc_v, dstc_v, aec_v)
    idx_hbms = (src_hbm, dst_hbm, ae_hbm)

    pltpu.sync_copy(asrc_hbm, asrc_v)
    pltpu.sync_copy(adst_hbm, adst_v)

    @pl.when(sid < NS - 1)
    def _():
        pltpu.sync_copy(zeros_hbm.at[pl.ds(sid * RPT, RPT)],
                        acc_sh.at[pl.ds(sid * RPT, RPT)])

    @pl.when(sid == NS - 1)
    def _():
        pltpu.sync_copy(zeros_hbm.at[pl.ds((NS - 1) * RPT, N - (NS - 1) * RPT)],
                        acc_sh.at[pl.ds((NS - 1) * RPT, N - (NS - 1) * RPT)])

    def zero_body(i, c):
        den_v[pl.ds(i * L, L)] = jnp.zeros((L,), jnp.float32)
        return c

    lax.fori_loop(0, N_ACC // L, zero_body, 0)
    plsc.subcore_barrier()

    ebase = wid * EPW

    def stage(S, sb):
        for hbm, buf in zip(idx_hbms, idx_bufs):
            pltpu.async_copy(hbm.at[pl.ds(ebase + S * KBRC, KBRC)],
                             buf.at[pl.ds(sb * KBRC, KBRC)], isem[sb])

    def stage_wait(sb):
        for hbm, buf in zip(idx_hbms, idx_bufs):
            pltpu.make_async_copy(hbm.at[pl.ds(0, KBRC)],
                                  buf.at[pl.ds(sb * KBRC, KBRC)],
                                  isem[sb]).wait()

    def gather(sb, kk, b):
        pltpu.async_copy(
            hext_hbm.at[srcc_v.at[pl.ds(sb * KBRC + kk * RC, RC)]],
            rows_v.at[b], gsem[b])

    def gather_wait(b):
        pltpu.make_async_copy(hext_hbm.at[srcc_v.at[pl.ds(0, RC)]],
                              rows_v.at[b], gsem[b]).wait()

    def scatter(b):
        pltpu.async_copy(rows_v.at[b], acc_sh.at[drow[b]], ssem[b], add=True)

    def scatter_wait(b):
        pltpu.make_async_copy(rows_v.at[0], acc_sh.at[dr0_v], ssem[b]).wait()

    stage(0, 0)
    stage_wait(0)
    gather(0, 0, 0)
    gather(0, 1, 1)

    def pair_body(S2, carry):
        for sp in (0, 1):
            S = S2 * 2 + sp
            for kk in range(KB):
                b = kk % 4
                bn2 = (kk + 2) % 4
                if kk >= 2:
                    scatter_wait(bn2)
                else:
                    @pl.when(S >= 1)
                    def _():
                        scatter_wait(bn2)
                if kk < KB - 2:
                    gather(sp, kk + 2, bn2)
                elif kk == KB - 2:
                    @pl.when(S < NSB - 1)
                    def _():
                        stage_wait(1 - sp)
                        gather(1 - sp, 0, bn2)
                else:
                    @pl.when(S < NSB - 1)
                    def _():
                        gather(1 - sp, 1, bn2)
                gather_wait(b)
                cb = sp * KBRC + kk * RC
                for g in range(RC // L):
                    idx_s = srcc_v[pl.ds(cb + g * L, L)]
                    idx_d = dstc_v[pl.ds(cb + g * L, L)]
                    drow[b][pl.ds(g * L, L)] = idx_d
                    a_s = plsc.load_gather(asrc_v, [idx_s])
                    a_d = plsc.load_gather(adst_v, [idx_d])
                    al = a_s + a_d + aec_v[pl.ds(cb + g * L, L)]
                    al = jnp.where(al >= 0.0, al, al * 0.2)
                    ex = jnp.exp(al)
                    ex_v[pl.ds(g * L, L)] = ex
                    plsc.addupdate_scatter(den_v, [idx_d], ex)

                @plsc.parallel_loop(0, RC, unroll=4)
                def scale_body(e):
                    exb = plsc.load_gather(ex_v, [jnp.broadcast_to(e, (L,))])
                    for k in range(HID // L):
                        rows_v[b, e, pl.ds(k * L, L)] = (
                            rows_v[b, e, pl.ds(k * L, L)] * exb)

                if kk == 2:
                    @pl.when(S < NSB - 1)
                    def _():
                        stage(S + 1, 1 - sp)
                scatter(b)
        return carry

    lax.fori_loop(0, NSB // 2, pair_body, 0)
    scatter_wait((NCHUNK - 2) % 4)
    scatter_wait((NCHUNK - 1) % 4)
    pltpu.sync_copy(den_v, den_hbm.at[wid])
    plsc.subcore_barrier()

    @pl.when(sid < NS - 1)
    def _():
        pltpu.sync_copy(acc_sh.at[pl.ds(sid * RPT, RPT)],
                        out_hbm.at[cid, pl.ds(sid * RPT, RPT)])

    @pl.when(sid == NS - 1)
    def _():
        pltpu.sync_copy(acc_sh.at[pl.ds((NS - 1) * RPT, N - (NS - 1) * RPT)],
                        out_hbm.at[cid, pl.ds((NS - 1) * RPT, N - (NS - 1) * RPT)])


@functools.cache
def _sc_gat():
    mesh = plsc.VectorSubcoreMesh(core_axis_name="c", subcore_axis_name="s",
                                  num_cores=NC, num_subcores=NS)
    return pl.kernel(
        _sc_body,
        out_type=(jax.ShapeDtypeStruct((NC, N_ACC, HEXT), jnp.float32),
                  jax.ShapeDtypeStruct((NW, N_ACC), jnp.float32)),
        mesh=mesh,
        compiler_params=pltpu.CompilerParams(needs_layout_passes=False),
        scratch_types=[
            pltpu.VMEM((N,), jnp.float32),
            pltpu.VMEM((N,), jnp.float32),
            pltpu.VMEM((2 * KBRC,), jnp.int32),
            pltpu.VMEM((2 * KBRC,), jnp.int32),
            pltpu.VMEM((2 * KBRC,), jnp.float32),
            pltpu.VMEM((RC,), jnp.int32),
            pltpu.VMEM((RC,), jnp.int32),
            pltpu.VMEM((RC,), jnp.int32),
            pltpu.VMEM((RC,), jnp.int32),
            pltpu.VMEM((RC,), jnp.float32),
            pltpu.VMEM((4, RC, HEXT), jnp.float32),
            pltpu.VMEM((N_ACC,), jnp.float32),
            pltpu.VMEM_SHARED((N, HEXT), jnp.float32),
        ] + [pltpu.SemaphoreType.DMA] * 10,
    )


def _pool_body(acc_ref, den_ref, n2g_ref, bgat_ref, out0_ref, p_ref, sums_sc, cnt_sc):
    i = pl.program_id(0)
    num = acc_ref[0] + acc_ref[1]
    den = jnp.sum(den_ref[:, pl.ds(i * PB, PB)], axis=0)[:, None]
    out = jnp.where(den > 0.0, num / (den + 1e-16), 0.0) + bgat_ref[...]

    @pl.when(i == 0)
    def _():
        out0_ref[...] = out[0:1, :]
        sums_sc[...] = jnp.zeros_like(sums_sc)
        cnt_sc[...] = jnp.zeros_like(cnt_sc)

    n2g = n2g_ref[0, 0, :]
    ind = (lax.broadcasted_iota(jnp.int32, (N_GRAPHS, PB), 0)
           == n2g[None, :]).astype(jnp.float32)
    sums_sc[...] += jnp.dot(ind, out, preferred_element_type=jnp.float32)
    cnt_sc[...] += jnp.broadcast_to(jnp.sum(ind, axis=-1)[:, None], (N_GRAPHS, HID))

    @pl.when(i == PBLK - 1)
    def _():
        p_ref[...] = sums_sc[...] / jnp.maximum(cnt_sc[...], 1.0)


def _pool_stage(acc, den, node2graph, b_gat):
    full = lambda shape: pl.BlockSpec(shape, lambda i: (0,) * len(shape))
    n2g = jnp.concatenate(
        [node2graph.astype(jnp.int32),
         jnp.full((N_ACC - N,), N_GRAPHS, jnp.int32)]).reshape(PBLK, 1, PB)
    return pl.pallas_call(
        _pool_body,
        grid=(PBLK,),
        in_specs=[
            pl.BlockSpec((NC, PB, HEXT), lambda i: (0, i, 0)),
            pl.BlockSpec((NW, N_ACC), lambda i: (0, 0)),
            pl.BlockSpec((1, 1, PB), lambda i: (i, 0, 0)),
            full((1, HID)),
        ],
        out_specs=[
            full((1, HID)),
            full((N_GRAPHS, HID)),
        ],
        out_shape=[
            jax.ShapeDtypeStruct((1, HID), jnp.float32),
            jax.ShapeDtypeStruct((N_GRAPHS, HID), jnp.float32),
        ],
        scratch_shapes=[
            pltpu.VMEM((N_GRAPHS, HID), jnp.float32),
            pltpu.VMEM((N_GRAPHS, HID), jnp.float32),
        ],
    )(acc, den, n2g, b_gat.reshape(1, HID))


def kernel(qa_emb, x, node_ids, node_types, node_scores, edge_index, edge_type,
           edge_attr, node2graph, W_qa, b_qa, W_nt, b_nt, W_e1, b_e1, W_e2, b_e2,
           W_lin, W_el, att_src, att_dst, att_edge, b_gat):
    h_ext, ad = _node_stage(x, node_types, node_scores, qa_emb, W_qa, b_qa,
                            W_nt, b_nt, W_lin, att_src, att_dst)
    a_src = ad[:, 0, :].reshape(N)
    a_dst = ad[:, 1, :].reshape(N)
    alpha_e = _edge_stage(edge_attr, W_e1, b_e1, W_e2, b_e2, W_el,
                          att_edge).reshape(E)
    pad = E_PAD - E
    src = jnp.concatenate(
        [edge_index[0].astype(jnp.int32),
         jnp.zeros((pad,), jnp.int32)])
    dst = jnp.concatenate(
        [edge_index[1].astype(jnp.int32),
         jnp.zeros((pad,), jnp.int32)])
    ae2 = jnp.concatenate(
        [alpha_e, jnp.full((pad,), -1e30, jnp.float32)])
    zeros = jnp.zeros((N, HEXT), jnp.float32)
    acc, den = _sc_gat()(src, dst, ae2, a_src, a_dst, h_ext, zeros)
    out0, p = _pool_stage(acc, den, node2graph, b_gat)
    return (out0.reshape(HID), p)


# speedup vs baseline: 1.7119x; 1.5551x over previous
import functools

import jax
import jax.numpy as jnp
from jax import lax
from jax.experimental import pallas as pl
from jax.experimental.pallas import tpu as pltpu
from jax.experimental.pallas import tpu_sc as plsc

N = 10000
E = 640000
QA_DIM = 1024
HID = 128
N_NTYPE = 4
N_ETYPE = 38
N_GRAPHS = 50
GC_IN = HID + HID // 2
EA_DIM = N_NTYPE + N_ETYPE + N_NTYPE

HEXT = HID
NC, NS, L = 2, 16, 16
NW = NC * NS
RC = 32
KB = 8
NSB = 80
NCHUNK = NSB * KB
EPW = NCHUNK * RC
E_PAD = NW * EPW
NB = 1000
NBLK = N // NB
EB = 6400
EBLK = E // EB
N_ACC = 10240
PB = 1024
PBLK = N_ACC // PB
KBRC = KB * RC
RPT = 624


def _node_body(x_ref, nt_ref, ns_ref, qa_ref, wqa_ref, bqa_ref, wnt_ref, bnt_ref,
               wlin_ref, asrc_w_ref, adst_w_ref, hext_ref, ad_ref):
    i = pl.program_id(0)
    x = x_ref[...]
    qa_row = jnp.dot(qa_ref[...], wqa_ref[...],
                     preferred_element_type=jnp.float32) + bqa_ref[...]
    row_ids = lax.broadcasted_iota(jnp.int32, (NB, 1), 0) + i * NB
    x = jnp.where(row_ids == 0, qa_row, x)
    xr = jax.nn.relu(x)
    nts = jnp.concatenate([nt_ref[...], ns_ref[...]], axis=-1)
    extras = jnp.dot(nts, wnt_ref[...], preferred_element_type=jnp.float32) + bnt_ref[...]
    er = jax.nn.relu(extras)
    h = (jnp.dot(xr, wlin_ref[0:HID, :], preferred_element_type=jnp.float32)
         + jnp.dot(er, wlin_ref[HID:GC_IN, :], preferred_element_type=jnp.float32))
    hext_ref[...] = h
    a_src = jnp.sum(h * asrc_w_ref[...], axis=-1)
    a_dst = jnp.sum(h * adst_w_ref[...], axis=-1)
    ad_ref[0, 0, :] = a_src
    ad_ref[0, 1, :] = a_dst


def _node_stage(x, node_types, node_scores, qa_emb, W_qa, b_qa, W_nt, b_nt, W_lin,
                att_src, att_dst):
    full = lambda shape: pl.BlockSpec(shape, lambda i: (0,) * len(shape))
    return pl.pallas_call(
        _node_body,
        grid=(NBLK,),
        in_specs=[
            pl.BlockSpec((NB, HID), lambda i: (i, 0)),
            pl.BlockSpec((NB, N_NTYPE), lambda i: (i, 0)),
            pl.BlockSpec((NB, 1), lambda i: (i, 0)),
            full((1, QA_DIM)),
            full((QA_DIM, HID)),
            full((1, HID)),
            full((N_NTYPE + 1, HID // 2)),
            full((1, HID // 2)),
            full((GC_IN, HID)),
            full((1, HID)),
            full((1, HID)),
        ],
        out_specs=[
            pl.BlockSpec((NB, HEXT), lambda i: (i, 0)),
            pl.BlockSpec((1, 2, NB), lambda i: (i, 0, 0)),
        ],
        out_shape=[
            jax.ShapeDtypeStruct((N, HEXT), jnp.float32),
            jax.ShapeDtypeStruct((NBLK, 2, NB), jnp.float32),
        ],
    )(x, node_types, node_scores, qa_emb.reshape(1, QA_DIM), W_qa,
      b_qa.reshape(1, HID), W_nt, b_nt.reshape(1, HID // 2), W_lin,
      att_src.reshape(1, HID), att_dst.reshape(1, HID))


def _edge_body(ea_ref, w1_ref, b1_ref, w2_ref, b2_ref, wel_ref, atte_ref, out_ref):
    t = jax.nn.relu(jnp.dot(w1_ref[...], ea_ref[...],
                            preferred_element_type=jnp.float32)
                    + b1_ref[...]).astype(jnp.bfloat16)
    sv = jax.nn.relu(jnp.dot(w2_ref[...], t,
                             preferred_element_type=jnp.float32)
                     + b2_ref[...]).astype(jnp.bfloat16)
    v_el = jnp.dot(wel_ref[...], atte_ref[...],
                   preferred_element_type=jnp.float32)
    velb = v_el.astype(jnp.bfloat16).reshape(1, HID)
    al = jnp.dot(velb, sv, preferred_element_type=jnp.float32)
    out_ref[...] = al.reshape(1, 1, EB)


def _edge_stage(edge_attr, W_e1, b_e1, W_e2, b_e2, W_el, att_edge):
    full = lambda shape: pl.BlockSpec(shape, lambda i: (0,) * len(shape))
    ea_t = edge_attr.astype(jnp.bfloat16).T
    return pl.pallas_call(
        _edge_body,
        grid=(EBLK,),
        in_specs=[
            pl.BlockSpec((EA_DIM, EB), lambda i: (0, i)),
            full((HID, EA_DIM)),
            full((HID, 1)),
            full((HID, HID)),
            full((HID, 1)),
            full((HID, HID)),
            full((HID, 1)),
        ],
        out_specs=pl.BlockSpec((1, 1, EB), lambda i: (i, 0, 0)),
        out_shape=jax.ShapeDtypeStruct((EBLK, 1, EB), jnp.float32),
    )(ea_t, W_e1.T.astype(jnp.bfloat16), b_e1.reshape(HID, 1),
      W_e2.T.astype(jnp.bfloat16), b_e2.reshape(HID, 1),
      W_el, att_edge.reshape(HID, 1))


def _sc_body(src_hbm, dst_hbm, ae_hbm, asrc_hbm, adst_hbm, hext_hbm, zeros_hbm,
             out_hbm, den_hbm, asrc_v, adst_v, srcc_v, dstc_v, aec_v,
             dr0_v, dr1_v, dr2_v, dr3_v, ex_v, rows_v, den_v, acc_sh,
             gsem0, gsem1, gsem2, gsem3, ssem0, ssem1, ssem2, ssem3,
             isem0, isem1):
    cid = lax.axis_index("c")
    sid = lax.axis_index("s")
    wid = cid * NS + sid
    gsem = (gsem0, gsem1, gsem2, gsem3)
    ssem = (ssem0, ssem1, ssem2, ssem3)
    isem = (isem0, isem1)
    drow = (dr0_v, dr1_v, dr2_v, dr3_v)
    idx_bufs = (srcc_v, dstc_v, aec_v)
    idx_hbms = (src_hbm, dst_hbm, ae_hbm)

    pltpu.sync_copy(asrc_hbm, asrc_v)
    pltpu.sync_copy(adst_hbm, adst_v)

    @pl.when(sid < NS - 1)
    def _():
        pltpu.sync_copy(zeros_hbm.at[pl.ds(sid * RPT, RPT)],
                        acc_sh.at[pl.ds(sid * RPT, RPT)])

    @pl.when(sid == NS - 1)
    def _():
        pltpu.sync_copy(zeros_hbm.at[pl.ds((NS - 1) * RPT, N - (NS - 1) * RPT)],
                        acc_sh.at[pl.ds((NS - 1) * RPT, N - (NS - 1) * RPT)])

    def zero_body(i, c):
        den_v[pl.ds(i * L, L)] = jnp.zeros((L,), jnp.float32)
        return c

    lax.fori_loop(0, N_ACC // L, zero_body, 0)
    plsc.subcore_barrier()

    ebase = wid * EPW

    def stage(S, sb):
        for hbm, buf in zip(idx_hbms, idx_bufs):
            pltpu.async_copy(hbm.at[pl.ds(ebase + S * KBRC, KBRC)],
                             buf.at[pl.ds(sb * KBRC, KBRC)], isem[sb])

    def stage_wait(sb):
        for hbm, buf in zip(idx_hbms, idx_bufs):
            pltpu.make_async_copy(hbm.at[pl.ds(0, KBRC)],
                                  buf.at[pl.ds(sb * KBRC, KBRC)],
                                  isem[sb]).wait()

    def gather(sb, kk, b):
        pltpu.async_copy(
            hext_hbm.at[srcc_v.at[pl.ds(sb * KBRC + kk * RC, RC)]],
            rows_v.at[b], gsem[b])

    def gather_wait(b):
        pltpu.make_async_copy(hext_hbm.at[srcc_v.at[pl.ds(0, RC)]],
                              rows_v.at[b], gsem[b]).wait()

    def scatter(b):
        pltpu.async_copy(rows_v.at[b], acc_sh.at[drow[b]], ssem[b], add=True)

    def scatter_wait(b):
        pltpu.make_async_copy(rows_v.at[0], acc_sh.at[dr0_v], ssem[b]).wait()

    stage(0, 0)
    stage_wait(0)
    gather(0, 0, 0)
    gather(0, 1, 1)

    def pair_body(S2, carry):
        for sp in (0, 1):
            S = S2 * 2 + sp
            for kk in range(KB):
                b = kk % 4
                bn2 = (kk + 2) % 4
                if kk >= 2:
                    scatter_wait(bn2)
                else:
                    @pl.when(S >= 1)
                    def _():
                        scatter_wait(bn2)
                if kk < KB - 2:
                    gather(sp, kk + 2, bn2)
                elif kk == KB - 2:
                    @pl.when(S < NSB - 1)
                    def _():
                        stage_wait(1 - sp)
                        gather(1 - sp, 0, bn2)
                else:
                    @pl.when(S < NSB - 1)
                    def _():
                        gather(1 - sp, 1, bn2)
                gather_wait(b)
                cb = sp * KBRC + kk * RC
                for g in range(RC // L):
                    idx_s = srcc_v[pl.ds(cb + g * L, L)]
                    idx_d = dstc_v[pl.ds(cb + g * L, L)]
                    drow[b][pl.ds(g * L, L)] = idx_d
                    a_s = plsc.load_gather(asrc_v, [idx_s])
                    a_d = plsc.load_gather(adst_v, [idx_d])
                    al = a_s + a_d + aec_v[pl.ds(cb + g * L, L)]
                    al = jnp.where(al >= 0.0, al, al * 0.2)
                    ex = jnp.exp(al)
                    ex_v[pl.ds(g * L, L)] = ex
                    plsc.addupdate_scatter(den_v, [idx_d], ex)

                @plsc.parallel_loop(0, RC, unroll=4)
                def scale_body(e):
                    exb = plsc.load_gather(ex_v, [jnp.broadcast_to(e, (L,))])
                    for k in range(HID // L):
                        rows_v[b, e, pl.ds(k * L, L)] = (
                            rows_v[b, e, pl.ds(k * L, L)] * exb)

                if kk == 2:
                    @pl.when(S < NSB - 1)
                    def _():
                        stage(S + 1, 1 - sp)
                scatter(b)
        return carry

    lax.fori_loop(0, NSB // 2, pair_body, 0)
    scatter_wait((NCHUNK - 2) % 4)
    scatter_wait((NCHUNK - 1) % 4)
    pltpu.sync_copy(den_v, den_hbm.at[wid])
    plsc.subcore_barrier()

    @pl.when(sid < NS - 1)
    def _():
        pltpu.sync_copy(acc_sh.at[pl.ds(sid * RPT, RPT)],
                        out_hbm.at[cid, pl.ds(sid * RPT, RPT)])

    @pl.when(sid == NS - 1)
    def _():
        pltpu.sync_copy(acc_sh.at[pl.ds((NS - 1) * RPT, N - (NS - 1) * RPT)],
                        out_hbm.at[cid, pl.ds((NS - 1) * RPT, N - (NS - 1) * RPT)])


@functools.cache
def _sc_gat():
    mesh = plsc.VectorSubcoreMesh(core_axis_name="c", subcore_axis_name="s",
                                  num_cores=NC, num_subcores=NS)
    return pl.kernel(
        _sc_body,
        out_type=(jax.ShapeDtypeStruct((NC, N_ACC, HEXT), jnp.float32),
                  jax.ShapeDtypeStruct((NW, N_ACC), jnp.float32)),
        mesh=mesh,
        compiler_params=pltpu.CompilerParams(needs_layout_passes=False),
        scratch_types=[
            pltpu.VMEM((N,), jnp.float32),
            pltpu.VMEM((N,), jnp.float32),
            pltpu.VMEM((2 * KBRC,), jnp.int32),
            pltpu.VMEM((2 * KBRC,), jnp.int32),
            pltpu.VMEM((2 * KBRC,), jnp.float32),
            pltpu.VMEM((RC,), jnp.int32),
            pltpu.VMEM((RC,), jnp.int32),
            pltpu.VMEM((RC,), jnp.int32),
            pltpu.VMEM((RC,), jnp.int32),
            pltpu.VMEM((RC,), jnp.float32),
            pltpu.VMEM((4, RC, HEXT), jnp.float32),
            pltpu.VMEM((N_ACC,), jnp.float32),
            pltpu.VMEM_SHARED((N, HEXT), jnp.float32),
        ] + [pltpu.SemaphoreType.DMA] * 10,
    )


def _pool_body(acc_ref, den_ref, n2g_ref, bgat_ref, out0_ref, p_ref, sums_sc, cnt_sc):
    i = pl.program_id(0)
    num = acc_ref[0] + acc_ref[1]
    den = jnp.sum(den_ref[:, pl.ds(i * PB, PB)], axis=0)[:, None]
    out = jnp.where(den > 0.0, num / (den + 1e-16), 0.0) + bgat_ref[...]

    @pl.when(i == 0)
    def _():
        out0_ref[...] = out[0:1, :]
        sums_sc[...] = jnp.zeros_like(sums_sc)
        cnt_sc[...] = jnp.zeros_like(cnt_sc)

    n2g = n2g_ref[0, 0, :]
    ind = (lax.broadcasted_iota(jnp.int32, (N_GRAPHS, PB), 0)
           == n2g[None, :]).astype(jnp.float32)
    sums_sc[...] += jnp.dot(ind, out, preferred_element_type=jnp.float32)
    cnt_sc[...] += jnp.broadcast_to(jnp.sum(ind, axis=-1)[:, None], (N_GRAPHS, HID))

    @pl.when(i == PBLK - 1)
    def _():
        p_ref[...] = sums_sc[...] / jnp.maximum(cnt_sc[...], 1.0)


def _pool_stage(acc, den, node2graph, b_gat):
    full = lambda shape: pl.BlockSpec(shape, lambda i: (0,) * len(shape))
    n2g = jnp.concatenate(
        [node2graph.astype(jnp.int32),
         jnp.full((N_ACC - N,), N_GRAPHS, jnp.int32)]).reshape(PBLK, 1, PB)
    return pl.pallas_call(
        _pool_body,
        grid=(PBLK,),
        in_specs=[
            pl.BlockSpec((NC, PB, HEXT), lambda i: (0, i, 0)),
            pl.BlockSpec((NW, N_ACC), lambda i: (0, 0)),
            pl.BlockSpec((1, 1, PB), lambda i: (i, 0, 0)),
            full((1, HID)),
        ],
        out_specs=[
            full((1, HID)),
            full((N_GRAPHS, HID)),
        ],
        out_shape=[
            jax.ShapeDtypeStruct((1, HID), jnp.float32),
            jax.ShapeDtypeStruct((N_GRAPHS, HID), jnp.float32),
        ],
        scratch_shapes=[
            pltpu.VMEM((N_GRAPHS, HID), jnp.float32),
            pltpu.VMEM((N_GRAPHS, HID), jnp.float32),
        ],
    )(acc, den, n2g, b_gat.reshape(1, HID))


def kernel(qa_emb, x, node_ids, node_types, node_scores, edge_index, edge_type,
           edge_attr, node2graph, W_qa, b_qa, W_nt, b_nt, W_e1, b_e1, W_e2, b_e2,
           W_lin, W_el, att_src, att_dst, att_edge, b_gat):
    h_ext, ad = _node_stage(x, node_types, node_scores, qa_emb, W_qa, b_qa,
                            W_nt, b_nt, W_lin, att_src, att_dst)
    a_src = ad[:, 0, :].reshape(N)
    a_dst = ad[:, 1, :].reshape(N)
    alpha_e = _edge_stage(edge_attr, W_e1, b_e1, W_e2, b_e2, W_el,
                          att_edge).reshape(E)
    pad = E_PAD - E
    src = jnp.concatenate(
        [edge_index[0].astype(jnp.int32),
         jnp.zeros((pad,), jnp.int32)])
    dst = jnp.concatenate(
        [edge_index[1].astype(jnp.int32),
         jnp.zeros((pad,), jnp.int32)])
    ae2 = jnp.concatenate(
        [alpha_e, jnp.full((pad,), -1e30, jnp.float32)])
    zeros = jnp.zeros((N, HEXT), jnp.float32)
    acc, den = _sc_gat()(src, dst, ae2, a_src, a_dst, h_ext, zeros)
    out0, p = _pool_stage(acc, den, node2graph, b_gat)
    return (out0.reshape(HID), p)
